# baseline (speedup 1.0000x reference)
import functools

import jax
import jax.numpy as jnp
from jax import lax
from jax.experimental import pallas as pl
from jax.experimental.pallas import tpu as pltpu
from jax.experimental.pallas import tpu_sc as plsc

_NC = 2
_NS = 16
_LANES = 16


_RING = 4


def _pick_chunk(epw):
    for c in range(128, 7, -8):
        if epw % c == 0 and epw // c >= 2 * _RING:
            return c
    raise ValueError(f"no valid chunk for {epw} edges per tile")


@functools.lru_cache(maxsize=None)
def _make_sc_scatter(n, d, e):
    n_tiles = _NC * _NS
    assert e % n_tiles == 0 and d % _LANES == 0
    epw = e // n_tiles
    chunk = _pick_chunk(epw)
    nchunk = epw // chunk
    rows_per_sub = -(-n // (8 * _NS)) * 8
    n_pad = rows_per_sub * _NS

    mesh = plsc.VectorSubcoreMesh(
        core_axis_name="c", subcore_axis_name="s",
        num_cores=_NC, num_subcores=_NS)

    @functools.partial(
        pl.kernel,
        mesh=mesh,
        out_type=jax.ShapeDtypeStruct((_NC * n_pad, d), jnp.float32),
        scratch_types=[
            [pltpu.VMEM((chunk,), jnp.int32) for _ in range(_RING)],
            [pltpu.VMEM((chunk,), jnp.int32) for _ in range(_RING)],
            [pltpu.VMEM((chunk,), jnp.float32) for _ in range(_RING)],
            [pltpu.VMEM((chunk, d), jnp.float32) for _ in range(_RING)],
            pltpu.VMEM_SHARED((n_pad, d), jnp.float32),
            [pltpu.SemaphoreType.DMA for _ in range(_RING)],
            [pltpu.SemaphoreType.DMA for _ in range(_RING)],
            [pltpu.SemaphoreType.DMA for _ in range(_RING)],
        ],
    )
    def sc_scatter(x_hbm, src_hbm, dst_hbm, ea_hbm, zeros_hbm, out_hbm,
                   src_v, dst_v, ea_v, rows_v, acc_sh, semg, semi, sems):
        c = lax.axis_index("c")
        s = lax.axis_index("s")
        tid = c * _NS + s
        r0 = s * rows_per_sub

        pltpu.sync_copy(zeros_hbm, acc_sh.at[pl.ds(r0, rows_per_sub)])
        plsc.subcore_barrier()

        ebase = tid * epw
        e_total = e

        def load_idx(ci_base, b):
            nb = jnp.minimum(ci_base, e_total - chunk)
            pltpu.async_copy(src_hbm.at[pl.ds(nb, chunk)], src_v[b], semi[b])
            pltpu.async_copy(dst_hbm.at[pl.ds(nb, chunk)], dst_v[b], semi[b])
            pltpu.async_copy(ea_hbm.at[pl.ds(nb, chunk)], ea_v[b], semi[b])
            return nb

        def drain_idx(nb, b):
            pltpu.make_async_copy(
                src_hbm.at[pl.ds(nb, chunk)], src_v[b], semi[b]).wait()
            pltpu.make_async_copy(
                dst_hbm.at[pl.ds(nb, chunk)], dst_v[b], semi[b]).wait()
            pltpu.make_async_copy(
                ea_hbm.at[pl.ds(nb, chunk)], ea_v[b], semi[b]).wait()

        def gather(b):
            pltpu.async_copy(x_hbm.at[src_v[b]], rows_v[b], semg[b])

        def wait_gather(b):
            pltpu.make_async_copy(
                x_hbm.at[src_v[b]], rows_v[b], semg[b]).wait()

        def scatter(b):
            pltpu.async_copy(
                rows_v[b], acc_sh.at[dst_v[b]], sems[b], add=True)

        def wait_scatter(b):
            pltpu.make_async_copy(
                rows_v[b], acc_sh.at[dst_v[b]], sems[b]).wait()

        def scale_rows(b):
            cur_g0, grp = -1, None
            for i in range(chunk):
                g0 = min((i // _LANES) * _LANES, chunk - _LANES)
                if g0 != cur_g0:
                    grp = ea_v[b][pl.ds(g0, _LANES)]
                    cur_g0 = g0
                scale = lax.gather(
                    grp, jnp.full((_LANES, 1), i - g0, jnp.int32),
                    lax.GatherDimensionNumbers(
                        offset_dims=(), collapsed_slice_dims=(0,),
                        start_index_map=(0,)),
                    (1,), mode=lax.GatherScatterMode.PROMISE_IN_BOUNDS)
                for g in range(d // _LANES):
                    sl = pl.ds(g * _LANES, _LANES)
                    rows_v[b][i, sl] = rows_v[b][i, sl] * scale

        p_len = _RING + (nchunk % _RING)
        nbs = [load_idx(ebase + q * chunk, q % _RING)
               for q in range(min(_RING, p_len))]
        for q in range(min(_RING, p_len)):
            drain_idx(nbs[q], q % _RING)
        for q in range(p_len):
            b = q % _RING
            if q >= _RING:
                wait_scatter(b)
                nb = load_idx(ebase + q * chunk, b)
                drain_idx(nb, b)
            gather(b)
            wait_gather(b)
            scale_rows(b)
            scatter(b)
        for q in range(p_len - _RING, p_len - 1):
            wait_scatter(q % _RING)
        for q in (p_len, p_len + 1):
            nb = load_idx(ebase + q * chunk, q % _RING)
            drain_idx(nb, q % _RING)
            gather(q % _RING)
        load_idx(ebase + (p_len + 2) * chunk, (p_len + 2) % _RING)

        def body(ci, carry):
            qbase = ebase + (p_len + ci * _RING) * chunk
            for j in range(_RING):
                b = (p_len + j) % _RING
                b2, b3 = (b + 2) % _RING, (b + 3) % _RING
                wait_scatter(b3)
                load_idx(qbase + (j + 3) * chunk, b3)
                wait_gather(b)
                nb2 = jnp.minimum(qbase + (j + 2) * chunk, e_total - chunk)
                drain_idx(nb2, b2)
                gather(b2)
                scale_rows(b)
                scatter(b)
            return carry

        lax.fori_loop(0, (nchunk - p_len) // _RING, body, 0)
        wait_gather(nchunk % _RING)
        wait_gather((nchunk + 1) % _RING)
        nb2 = jnp.minimum(ebase + (nchunk + 2) * chunk, e_total - chunk)
        drain_idx(nb2, (nchunk + 2) % _RING)
        wait_scatter((nchunk - 1) % _RING)

        plsc.subcore_barrier()
        pltpu.sync_copy(acc_sh.at[pl.ds(r0, rows_per_sub)],
                        out_hbm.at[pl.ds(c * n_pad + r0, rows_per_sub)])

    return sc_scatter, n_pad, rows_per_sub


def _tc_linear(p, w, b, n):
    dout = w.shape[0]

    def mm(p_ref, w_ref, b_ref, o_ref):
        ft = p_ref[0, :n] + p_ref[1, :n]
        o_ref[...] = lax.dot_general(
            ft, w_ref[...], (((1,), (1,)), ((), ())),
            preferred_element_type=jnp.float32) + b_ref[...]

    return pl.pallas_call(
        mm,
        out_shape=jax.ShapeDtypeStruct((n, dout), jnp.float32),
    )(p, w, b.reshape(1, dout))


def kernel(x, edge_index, edge_attr, W, b):
    n, d = x.shape
    e = edge_index.shape[1]
    src = edge_index[0].astype(jnp.int32)
    dst = edge_index[1].astype(jnp.int32)
    ea = edge_attr.reshape(e).astype(jnp.float32)
    sc, n_pad, rows_per_sub = _make_sc_scatter(n, d, e)
    zeros = jnp.zeros((rows_per_sub, d), jnp.float32)
    p = sc(x, src, dst, ea, zeros)
    return _tc_linear(p.reshape(_NC, n_pad, d), W, b, n)


# speedup vs baseline: 1.1709x; 1.1709x over previous
import functools

import jax
import jax.numpy as jnp
from jax import lax
from jax.experimental import pallas as pl
from jax.experimental.pallas import tpu as pltpu
from jax.experimental.pallas import tpu_sc as plsc

_NC = 2
_NS = 16
_LANES = 16


_RING = 4


def _pick_chunk(epw):
    for c in range(128, 7, -8):
        if epw % c == 0 and epw // c >= 2 * _RING:
            return c
    raise ValueError(f"no valid chunk for {epw} edges per tile")


@functools.lru_cache(maxsize=None)
def _make_sc_scatter(n, d, e):
    n_tiles = _NC * _NS
    assert e % n_tiles == 0 and d % _LANES == 0
    epw = e // n_tiles
    chunk = _pick_chunk(epw)
    nchunk = epw // chunk
    rows_per_sub = -(-n // (8 * _NS)) * 8
    n_pad = rows_per_sub * _NS

    mesh = plsc.VectorSubcoreMesh(
        core_axis_name="c", subcore_axis_name="s",
        num_cores=_NC, num_subcores=_NS)

    @functools.partial(
        pl.kernel,
        mesh=mesh,
        out_type=jax.ShapeDtypeStruct((_NC * n_pad, d), jnp.float32),
        scratch_types=[
            [pltpu.VMEM((chunk,), jnp.int32) for _ in range(_RING)],
            [pltpu.VMEM((chunk,), jnp.int32) for _ in range(_RING)],
            [pltpu.VMEM((chunk,), jnp.float32) for _ in range(_RING)],
            [pltpu.VMEM((chunk, d), jnp.float32) for _ in range(_RING)],
            pltpu.VMEM_SHARED((n_pad, d), jnp.float32),
            [pltpu.SemaphoreType.DMA for _ in range(_RING)],
            [pltpu.SemaphoreType.DMA for _ in range(_RING)],
            [pltpu.SemaphoreType.DMA for _ in range(_RING)],
            [pltpu.SemaphoreType.DMA for _ in range(_RING)],
        ],
    )
    def sc_scatter(x_hbm, src_hbm, dst_hbm, ea_hbm, zeros_hbm, out_hbm,
                   src_v, dst_v, ea_v, rows_v, acc_sh, semg, semi, semd,
                   sems):
        c = lax.axis_index("c")
        s = lax.axis_index("s")
        tid = c * _NS + s
        r0 = s * rows_per_sub

        pltpu.sync_copy(zeros_hbm, acc_sh.at[pl.ds(r0, rows_per_sub)])
        plsc.subcore_barrier()

        ebase = tid * epw
        e_total = e

        def _clamp(ci_base):
            return jnp.minimum(ci_base, e_total - chunk)

        def load_src_ea(ci_base, b):
            nb = _clamp(ci_base)
            pltpu.async_copy(src_hbm.at[pl.ds(nb, chunk)], src_v[b], semi[b])
            pltpu.async_copy(ea_hbm.at[pl.ds(nb, chunk)], ea_v[b], semi[b])

        def drain_src_ea(ci_base, b):
            nb = _clamp(ci_base)
            pltpu.make_async_copy(
                src_hbm.at[pl.ds(nb, chunk)], src_v[b], semi[b]).wait()
            pltpu.make_async_copy(
                ea_hbm.at[pl.ds(nb, chunk)], ea_v[b], semi[b]).wait()

        def load_dst(ci_base, b):
            nb = _clamp(ci_base)
            pltpu.async_copy(dst_hbm.at[pl.ds(nb, chunk)], dst_v[b], semd[b])

        def drain_dst(ci_base, b):
            nb = _clamp(ci_base)
            pltpu.make_async_copy(
                dst_hbm.at[pl.ds(nb, chunk)], dst_v[b], semd[b]).wait()

        def gather(b):
            pltpu.async_copy(x_hbm.at[src_v[b]], rows_v[b], semg[b])

        def wait_gather(b):
            pltpu.make_async_copy(
                x_hbm.at[src_v[b]], rows_v[b], semg[b]).wait()

        def scatter(b):
            pltpu.async_copy(
                rows_v[b], acc_sh.at[dst_v[b]], sems[b], add=True)

        def wait_scatter(b):
            pltpu.make_async_copy(
                rows_v[b], acc_sh.at[dst_v[b]], sems[b]).wait()

        def scale_rows(b):
            cur_g0, grp = -1, None
            for i in range(chunk):
                g0 = min((i // _LANES) * _LANES, chunk - _LANES)
                if g0 != cur_g0:
                    grp = ea_v[b][pl.ds(g0, _LANES)]
                    cur_g0 = g0
                scale = lax.gather(
                    grp, jnp.full((_LANES, 1), i - g0, jnp.int32),
                    lax.GatherDimensionNumbers(
                        offset_dims=(), collapsed_slice_dims=(0,),
                        start_index_map=(0,)),
                    (1,), mode=lax.GatherScatterMode.PROMISE_IN_BOUNDS)
                for g in range(d // _LANES):
                    sl = pl.ds(g * _LANES, _LANES)
                    rows_v[b][i, sl] = rows_v[b][i, sl] * scale

        p_len = _RING + (nchunk % _RING)
        for q in range(min(_RING, p_len)):
            load_src_ea(ebase + q * chunk, q)
            load_dst(ebase + q * chunk, q)
        for q in range(p_len):
            b = q % _RING
            base = ebase + q * chunk
            if q >= _RING:
                wait_scatter(b)
                load_src_ea(base, b)
                load_dst(base, b)
            drain_src_ea(base, b)
            drain_dst(base, b)
            gather(b)
            wait_gather(b)
            scale_rows(b)
            scatter(b)
        for q in (p_len - _RING, p_len - _RING + 1):
            wait_scatter(q % _RING)
        for q in (p_len, p_len + 1):
            b = q % _RING
            load_src_ea(ebase + q * chunk, b)
            load_dst(ebase + q * chunk, b)
            drain_src_ea(ebase + q * chunk, b)
            gather(b)
        load_src_ea(ebase + (p_len + 2) * chunk, (p_len + 2) % _RING)

        def body(ci, carry):
            qbase = ebase + (p_len + ci * _RING) * chunk
            for j in range(_RING):
                b = (p_len + j) % _RING
                b2, b3 = (b + 2) % _RING, (b + 3) % _RING
                wait_scatter(b2)
                load_dst(qbase + (j + 2) * chunk, b2)
                load_src_ea(qbase + (j + 3) * chunk, b3)
                wait_gather(b)
                drain_src_ea(qbase + (j + 2) * chunk, b2)
                gather(b2)
                scale_rows(b)
                drain_dst(qbase + j * chunk, b)
                scatter(b)
            return carry

        lax.fori_loop(0, (nchunk - p_len) // _RING, body, 0)
        wait_gather(nchunk % _RING)
        wait_gather((nchunk + 1) % _RING)
        drain_src_ea(ebase + (nchunk + 2) * chunk, (nchunk + 2) % _RING)
        drain_dst(ebase + nchunk * chunk, nchunk % _RING)
        drain_dst(ebase + (nchunk + 1) * chunk, (nchunk + 1) % _RING)
        for q in range(nchunk - 2, nchunk):
            wait_scatter(q % _RING)

        plsc.subcore_barrier()
        pltpu.sync_copy(acc_sh.at[pl.ds(r0, rows_per_sub)],
                        out_hbm.at[pl.ds(c * n_pad + r0, rows_per_sub)])

    return sc_scatter, n_pad, rows_per_sub


def _tc_linear(p, w, b, n):
    dout = w.shape[0]

    def mm(p_ref, w_ref, b_ref, o_ref):
        ft = p_ref[0, :n] + p_ref[1, :n]
        o_ref[...] = lax.dot_general(
            ft, w_ref[...], (((1,), (1,)), ((), ())),
            preferred_element_type=jnp.float32) + b_ref[...]

    return pl.pallas_call(
        mm,
        out_shape=jax.ShapeDtypeStruct((n, dout), jnp.float32),
    )(p, w, b.reshape(1, dout))


def kernel(x, edge_index, edge_attr, W, b):
    n, d = x.shape
    e = edge_index.shape[1]
    src = edge_index[0].astype(jnp.int32)
    dst = edge_index[1].astype(jnp.int32)
    ea = edge_attr.reshape(e).astype(jnp.float32)
    sc, n_pad, rows_per_sub = _make_sc_scatter(n, d, e)
    zeros = jnp.zeros((rows_per_sub, d), jnp.float32)
    p = sc(x, src, dst, ea, zeros)
    return _tc_linear(p.reshape(_NC, n_pad, d), W, b, n)
